# Initial kernel scaffold; baseline (speedup 1.0000x reference)
import functools

import jax
import jax.numpy as jnp
from jax import lax
from jax.experimental import pallas as pl
from jax.experimental.pallas import tpu as pltpu
from jax.experimental.pallas import tpu_sc as plsc

N_NODES = 10000
N_EDGES = 320000
F = 128

NC = 2
NS = 16
L = 16
NW = NC * NS
CH = 128
CPW = 80
EPW = CPW * CH
E_PAD = NW * EPW
RPW = N_NODES // NS


def _sc_aggregate(x, src, dst, w):
    mesh = plsc.VectorSubcoreMesh(core_axis_name="c", subcore_axis_name="s")

    @functools.partial(
        pl.kernel,
        out_type=jax.ShapeDtypeStruct((NC, N_NODES, F), jnp.float32),
        mesh=mesh,
        scratch_types=[
            pltpu.VMEM((EPW,), jnp.int32),
            pltpu.VMEM((CPW, CH), jnp.int32),
            pltpu.VMEM((CPW, CH), jnp.float32),
            pltpu.VMEM((CH, F), jnp.float32),
            pltpu.VMEM((CH, F), jnp.float32),
            pltpu.VMEM_SHARED((N_NODES, F), jnp.float32),
            pltpu.SemaphoreType.DMA,
        ],
    )
    def body(x_hbm, src_hbm, dst_hbm, w_hbm, out_hbm,
             src_v, dst_v, w_v, rows_v, stage_v, acc_sh, sem):
        cid = lax.axis_index("c")
        sid = lax.axis_index("s")
        wid = cid * NS + sid

        zero = jnp.zeros((L,), jnp.float32)

        def zrow(i, carry):
            for q in range(F // L):
                stage_v[i, pl.ds(q * L, L)] = zero
            return carry

        lax.fori_loop(0, CH, zrow, 0)

        base_rows = sid * RPW
        r = 0
        while r < RPW:
            n = min(CH, RPW - r)
            pltpu.sync_copy(stage_v.at[pl.ds(0, n)],
                            acc_sh.at[pl.ds(base_rows + r, n)])
            r += n
        plsc.subcore_barrier()

        pltpu.sync_copy(src_hbm.at[wid], src_v)
        pltpu.sync_copy(dst_hbm.at[wid], dst_v)
        pltpu.sync_copy(w_hbm.at[wid], w_v)

        def chunk(j, carry):
            idx = src_v.at[pl.ds(pl.multiple_of(j * CH, CH), CH)]
            pltpu.async_copy(x_hbm.at[idx], rows_v, sem).wait()

            def scale16(k, c2):
                wv = w_v[j, pl.ds(pl.multiple_of(k * L, L), L)]
                for e in range(L):
                    s = jnp.take(wv, jnp.full((L,), e, jnp.int32),
                                 mode="promise_in_bounds")
                    row = k * L + e
                    for q in range(F // L):
                        sl = pl.ds(q * L, L)
                        rows_v[row, sl] = rows_v[row, sl] * s
                return c2

            lax.fori_loop(0, CH // L, scale16, 0)
            pltpu.sync_copy(rows_v, acc_sh.at[dst_v.at[j]], add=True)
            return carry

        lax.fori_loop(0, CPW, chunk, 0)
        plsc.subcore_barrier()

        r = 0
        while r < RPW:
            n = min(CH, RPW - r)
            pltpu.sync_copy(acc_sh.at[pl.ds(base_rows + r, n)], stage_v.at[pl.ds(0, n)])
            pltpu.sync_copy(stage_v.at[pl.ds(0, n)],
                            out_hbm.at[cid, pl.ds(base_rows + r, n)])
            r += n

    return body(x, src, dst, w)


def _tc_combine_matmul(y, W, b):
    blk = 1000

    def body(y_ref, w_ref, b_ref, o_ref):
        ys = y_ref[0] + y_ref[1]
        o_ref[...] = (jnp.dot(ys, w_ref[...], preferred_element_type=jnp.float32)
                      + b_ref[...])

    return pl.pallas_call(
        body,
        grid=(N_NODES // blk,),
        in_specs=[
            pl.BlockSpec((NC, blk, F), lambda i: (0, i, 0)),
            pl.BlockSpec((F, F), lambda i: (0, 0)),
            pl.BlockSpec((1, F), lambda i: (0, 0)),
        ],
        out_specs=pl.BlockSpec((blk, F), lambda i: (i, 0)),
        out_shape=jax.ShapeDtypeStruct((N_NODES, F), jnp.float32),
    )(y, W, b.reshape(1, F))


def kernel(x, edge_index, edge_weight, W, b):
    src = edge_index[0].astype(jnp.int32)
    dst = edge_index[1].astype(jnp.int32)
    w = edge_weight.astype(jnp.float32)

    pad = E_PAD - N_EDGES
    src_p = jnp.pad(src, (0, pad)).reshape(NW, EPW)
    dst_p = jnp.pad(dst, (0, pad)).reshape(NW, CPW, CH)
    w_p = jnp.pad(w, (0, pad)).reshape(NW, CPW, CH)

    y = _sc_aggregate(x, src_p, dst_p, w_p)
    return _tc_combine_matmul(y, W, b)


# speedup vs baseline: 3.7408x; 3.7408x over previous
import functools

import jax
import jax.numpy as jnp
from jax import lax
from jax.experimental import pallas as pl
from jax.experimental.pallas import tpu as pltpu
from jax.experimental.pallas import tpu_sc as plsc

N_NODES = 10000
N_EDGES = 320000
F = 128

NC = 2
NS = 16
L = 16
NW = NC * NS
CH = 128
CPW = 80
EPW = CPW * CH
E_PAD = NW * EPW

_SLABS = [(t * 624, 624) for t in range(NS - 1)] + [((NS - 1) * 624, 640)]


def _slab_chunks(off, ln):
    out = []
    r = 0
    while r < ln:
        n = min(CH, ln - r)
        out.append((off + r, n))
        r += n
    return out


def _sc_aggregate(x, edata, ew):
    mesh = plsc.VectorSubcoreMesh(core_axis_name="c", subcore_axis_name="s")

    @functools.partial(
        pl.kernel,
        out_type=jax.ShapeDtypeStruct((NC, N_NODES, F), jnp.float32),
        mesh=mesh,
        scratch_types=[
            pltpu.VMEM((CPW, 2, CH), jnp.int32),
            pltpu.VMEM((CPW, CH), jnp.float32),
            pltpu.VMEM((CH, F), jnp.float32),
            pltpu.VMEM_SHARED((N_NODES, F), jnp.float32),
            pltpu.SemaphoreType.DMA,
        ],
    )
    def body(x_hbm, ed_hbm, ew_hbm, out_hbm, ed_v, w_v, rows_v, acc_sh, sem):
        cid = lax.axis_index("c")
        sid = lax.axis_index("s")
        wid = cid * NS + sid

        zero = jnp.zeros((L,), jnp.float32)

        def zrow(i, carry):
            for q in range(F // L):
                rows_v[i, pl.ds(q * L, L)] = zero
            return carry

        lax.fori_loop(0, CH, zrow, 0)

        for t, (off, ln) in enumerate(_SLABS):
            @pl.when(sid == t)
            def _():
                for o, n in _slab_chunks(off, ln):
                    pltpu.sync_copy(rows_v.at[pl.ds(0, n)],
                                    acc_sh.at[pl.ds(o, n)])
        plsc.subcore_barrier()

        pltpu.sync_copy(ed_hbm.at[wid], ed_v)
        pltpu.sync_copy(ew_hbm.at[wid], w_v)

        dnums = lax.GatherDimensionNumbers(
            offset_dims=(), collapsed_slice_dims=(0,), start_index_map=(0,))

        def chunk(j, carry):
            pltpu.async_copy(x_hbm.at[ed_v.at[j, 0]], rows_v, sem).wait()

            def scale16(k, c2):
                wv = w_v[j, pl.ds(pl.multiple_of(k * L, L), L)]
                for e in range(L):
                    s = lax.gather(wv, jnp.full((L, 1), e, jnp.int32),
                                   dnums, slice_sizes=(1,),
                                   mode=lax.GatherScatterMode.PROMISE_IN_BOUNDS)
                    row = k * L + e
                    for q in range(F // L):
                        sl = pl.ds(q * L, L)
                        rows_v[row, sl] = rows_v[row, sl] * s
                return c2

            lax.fori_loop(0, CH // L, scale16, 0)
            pltpu.sync_copy(rows_v, acc_sh.at[ed_v.at[j, 1]], add=True)
            return carry

        lax.fori_loop(0, CPW, chunk, 0)
        plsc.subcore_barrier()

        for t, (off, ln) in enumerate(_SLABS):
            @pl.when(sid == t)
            def _():
                for o, n in _slab_chunks(off, ln):
                    pltpu.sync_copy(acc_sh.at[pl.ds(o, n)],
                                    rows_v.at[pl.ds(0, n)])
                    pltpu.sync_copy(rows_v.at[pl.ds(0, n)],
                                    out_hbm.at[cid, pl.ds(o, n)])

    return body(x, edata, ew)


def _tc_combine_matmul(y, W, b):
    blk = 1000

    def body(y_ref, w_ref, b_ref, o_ref):
        ys = y_ref[0] + y_ref[1]
        o_ref[...] = (jnp.dot(ys, w_ref[...], preferred_element_type=jnp.float32)
                      + b_ref[...])

    return pl.pallas_call(
        body,
        grid=(N_NODES // blk,),
        in_specs=[
            pl.BlockSpec((NC, blk, F), lambda i: (0, i, 0)),
            pl.BlockSpec((F, F), lambda i: (0, 0)),
            pl.BlockSpec((1, F), lambda i: (0, 0)),
        ],
        out_specs=pl.BlockSpec((blk, F), lambda i: (i, 0)),
        out_shape=jax.ShapeDtypeStruct((N_NODES, F), jnp.float32),
    )(y, W, b.reshape(1, F))


def kernel(x, edge_index, edge_weight, W, b):
    src = edge_index[0].astype(jnp.int32)
    dst = edge_index[1].astype(jnp.int32)

    pad = E_PAD - N_EDGES
    edata = jnp.stack([
        jnp.pad(src, (0, pad)),
        jnp.pad(dst, (0, pad)),
    ], axis=0).reshape(2, NW, CPW, CH).transpose(1, 2, 0, 3)
    ew = jnp.pad(edge_weight.astype(jnp.float32),
                 (0, pad)).reshape(NW, CPW, CH)

    y = _sc_aggregate(x, edata, ew)
    return _tc_combine_matmul(y, W, b)
